# Initial kernel scaffold; baseline (speedup 1.0000x reference)
import jax
import jax.numpy as jnp
from jax.experimental import pallas as pl


def kernel(input, binding_keys, binding_values, Wq, bq, Wr, br):
    raise NotImplementedError("write your pallas kernel here")


# speedup vs baseline: 29.0710x; 29.0710x over previous
import functools

import jax
import jax.numpy as jnp
from jax.experimental import pallas as pl

_TOP_K = 8


def _row_tile(n, target):
    t = min(n, target)
    while n % t:
        t -= 1
    return t


def _norm_kernel(k_ref, o_ref):
    k = k_ref[0]
    n = jnp.sqrt(jnp.sum(k * k, axis=-1, keepdims=True))
    o_ref[0] = k / n


def _qres_kernel(x_ref, wq_ref, bq_ref, wr_ref, br_ref, q_ref, res_ref):
    q = jax.lax.dot_general(
        x_ref[...], wq_ref[...], (((1,), (1,)), ((), ())),
        preferred_element_type=jnp.float32) + bq_ref[...]
    q_ref[...] = q
    res_ref[...] = jax.lax.dot_general(
        q, wr_ref[...], (((1,), (1,)), ((), ())),
        preferred_element_type=jnp.float32) + br_ref[...]


def _main_kernel(q_ref, sk_ref, v_ref, res_ref, o_ref, *, n_keys):
    q = q_ref[...]
    sk = sk_ref[0]
    scores = jax.lax.dot_general(
        q, sk, (((1,), (1,)), ((), ())),
        preferred_element_type=jnp.float32)
    tr = scores.shape[0]
    col = jax.lax.broadcasted_iota(jnp.int32, (tr, n_keys), 1)
    neg = jnp.float32(-jnp.inf)
    m = scores
    denom = jnp.zeros((tr, 1), jnp.float32)
    s_top = None
    for i in range(_TOP_K):
        cur = jnp.max(m, axis=1, keepdims=True)
        if i == 0:
            s_top = cur
        denom = denom + jnp.exp(cur - s_top)
        eq = m == cur
        midx = jnp.where(eq, col, n_keys)
        amin = jnp.min(midx, axis=1, keepdims=True)
        hot = col == amin
        m = jnp.where(hot, neg, m)
    w = jnp.where(m == neg, jnp.exp(scores - s_top), 0.0) / denom
    out = jax.lax.dot_general(
        w, v_ref[0], (((1,), (0,)), ((), ())),
        preferred_element_type=jnp.float32)
    o_ref[0] = out + res_ref[...]


def kernel(input, binding_keys, binding_values, Wq, bq, Wr, br):
    prefix = input.shape[:-1]
    d_in = input.shape[-1]
    n_slots, n_keys, k_dim = binding_keys.shape
    v_dim = binding_values.shape[-1]
    x = input.reshape(-1, d_in)
    bs = x.shape[0]

    sk = pl.pallas_call(
        _norm_kernel,
        grid=(n_slots,),
        in_specs=[pl.BlockSpec((1, n_keys, k_dim), lambda s: (s, 0, 0))],
        out_specs=pl.BlockSpec((1, n_keys, k_dim), lambda s: (s, 0, 0)),
        out_shape=jax.ShapeDtypeStruct((n_slots, n_keys, k_dim), jnp.float32),
    )(binding_keys)

    tq = _row_tile(bs, 512)
    q, res = pl.pallas_call(
        _qres_kernel,
        grid=(bs // tq,),
        in_specs=[
            pl.BlockSpec((tq, d_in), lambda r: (r, 0)),
            pl.BlockSpec((k_dim, d_in), lambda r: (0, 0)),
            pl.BlockSpec((1, k_dim), lambda r: (0, 0)),
            pl.BlockSpec((v_dim, k_dim), lambda r: (0, 0)),
            pl.BlockSpec((1, v_dim), lambda r: (0, 0)),
        ],
        out_specs=[
            pl.BlockSpec((tq, k_dim), lambda r: (r, 0)),
            pl.BlockSpec((tq, v_dim), lambda r: (r, 0)),
        ],
        out_shape=[
            jax.ShapeDtypeStruct((bs, k_dim), jnp.float32),
            jax.ShapeDtypeStruct((bs, v_dim), jnp.float32),
        ],
    )(x, Wq, bq.reshape(1, -1), Wr, br.reshape(1, -1))

    tr = _row_tile(bs, 256)
    out = pl.pallas_call(
        functools.partial(_main_kernel, n_keys=n_keys),
        grid=(n_slots, bs // tr),
        in_specs=[
            pl.BlockSpec((tr, k_dim), lambda s, r: (r, 0)),
            pl.BlockSpec((1, n_keys, k_dim), lambda s, r: (s, 0, 0)),
            pl.BlockSpec((1, n_keys, v_dim), lambda s, r: (s, 0, 0)),
            pl.BlockSpec((tr, v_dim), lambda s, r: (r, 0)),
        ],
        out_specs=pl.BlockSpec((1, tr, v_dim), lambda s, r: (s, r, 0)),
        out_shape=jax.ShapeDtypeStruct((n_slots, bs, v_dim), jnp.float32),
    )(q, sk, binding_values, res)

    return jnp.transpose(out, (1, 0, 2)).reshape(prefix + (n_slots, v_dim))


# speedup vs baseline: 46.2481x; 1.5909x over previous
import functools

import jax
import jax.numpy as jnp
from jax.experimental import pallas as pl

_TOP_K = 8


def _row_tile(n, target):
    t = min(n, target)
    while n % t:
        t -= 1
    return t


def _norm_kernel(k_ref, o_ref):
    k = k_ref[0]
    n = jnp.sqrt(jnp.sum(k * k, axis=-1, keepdims=True))
    o_ref[0] = (k / n).astype(jnp.bfloat16)


def _qres_kernel(x_ref, wq_ref, bq_ref, wr_ref, br_ref, q16_ref, res_ref):
    q = jax.lax.dot_general(
        x_ref[...], wq_ref[...], (((1,), (1,)), ((), ())),
        preferred_element_type=jnp.float32) + bq_ref[...]
    q16_ref[...] = q.astype(jnp.bfloat16)
    res_ref[...] = jax.lax.dot_general(
        q, wr_ref[...], (((1,), (1,)), ((), ())),
        preferred_element_type=jnp.float32) + br_ref[...]


def _main_kernel(q_ref, sk_ref, v_ref, res_ref, o_ref):
    q = q_ref[...]
    sk = sk_ref[0]
    scores = jax.lax.dot_general(
        q, sk, (((1,), (1,)), ((), ())),
        preferred_element_type=jnp.float32)
    neg = jnp.float32(-jnp.inf)
    m = scores
    s_top = None
    for i in range(_TOP_K):
        cur = jnp.max(m, axis=1, keepdims=True)
        if i == 0:
            s_top = cur
        m = jnp.where(m == cur, neg, m)
    ew = jnp.where(m == neg, jnp.exp(scores - s_top), 0.0)
    denom = jnp.sum(ew, axis=1, keepdims=True)
    out = jax.lax.dot_general(
        ew.astype(jnp.bfloat16), v_ref[0], (((1,), (0,)), ((), ())),
        preferred_element_type=jnp.float32)
    o_ref[0] = out / denom + res_ref[...]


def kernel(input, binding_keys, binding_values, Wq, bq, Wr, br):
    prefix = input.shape[:-1]
    d_in = input.shape[-1]
    n_slots, n_keys, k_dim = binding_keys.shape
    v_dim = binding_values.shape[-1]
    x = input.reshape(-1, d_in)
    bs = x.shape[0]
    values16 = binding_values.astype(jnp.bfloat16)

    sk = pl.pallas_call(
        _norm_kernel,
        grid=(n_slots,),
        in_specs=[pl.BlockSpec((1, n_keys, k_dim), lambda s: (s, 0, 0))],
        out_specs=pl.BlockSpec((1, n_keys, k_dim), lambda s: (s, 0, 0)),
        out_shape=jax.ShapeDtypeStruct((n_slots, n_keys, k_dim), jnp.bfloat16),
    )(binding_keys)

    tq = _row_tile(bs, 512)
    q16, res = pl.pallas_call(
        _qres_kernel,
        grid=(bs // tq,),
        in_specs=[
            pl.BlockSpec((tq, d_in), lambda r: (r, 0)),
            pl.BlockSpec((k_dim, d_in), lambda r: (0, 0)),
            pl.BlockSpec((1, k_dim), lambda r: (0, 0)),
            pl.BlockSpec((v_dim, k_dim), lambda r: (0, 0)),
            pl.BlockSpec((1, v_dim), lambda r: (0, 0)),
        ],
        out_specs=[
            pl.BlockSpec((tq, k_dim), lambda r: (r, 0)),
            pl.BlockSpec((tq, v_dim), lambda r: (r, 0)),
        ],
        out_shape=[
            jax.ShapeDtypeStruct((bs, k_dim), jnp.bfloat16),
            jax.ShapeDtypeStruct((bs, v_dim), jnp.float32),
        ],
    )(x, Wq, bq.reshape(1, -1), Wr, br.reshape(1, -1))

    tr = _row_tile(bs, 256)
    out = pl.pallas_call(
        _main_kernel,
        grid=(n_slots, bs // tr),
        in_specs=[
            pl.BlockSpec((tr, k_dim), lambda s, r: (r, 0)),
            pl.BlockSpec((1, n_keys, k_dim), lambda s, r: (s, 0, 0)),
            pl.BlockSpec((1, n_keys, v_dim), lambda s, r: (s, 0, 0)),
            pl.BlockSpec((tr, v_dim), lambda s, r: (r, 0)),
        ],
        out_specs=pl.BlockSpec((1, tr, v_dim), lambda s, r: (s, r, 0)),
        out_shape=jax.ShapeDtypeStruct((n_slots, bs, v_dim), jnp.float32),
    )(q16, sk, values16, res)

    return jnp.transpose(out, (1, 0, 2)).reshape(prefix + (n_slots, v_dim))


# speedup vs baseline: 55.3411x; 1.1966x over previous
import functools

import jax
import jax.numpy as jnp
from jax.experimental import pallas as pl

_TOP_K = 8


def _row_tile(n, target):
    t = min(n, target)
    while n % t:
        t -= 1
    return t


def _norm_kernel(k_ref, o_ref):
    k = k_ref[0]
    n = jnp.sqrt(jnp.sum(k * k, axis=-1, keepdims=True))
    o_ref[0] = (k / n).astype(jnp.bfloat16)


def _qres_kernel(x_ref, wq_ref, bq_ref, wr_ref, br_ref, q16_ref, res_ref):
    q = jax.lax.dot_general(
        x_ref[...], wq_ref[...], (((1,), (1,)), ((), ())),
        preferred_element_type=jnp.float32) + bq_ref[...]
    q16_ref[...] = q.astype(jnp.bfloat16)
    res_ref[...] = jax.lax.dot_general(
        q, wr_ref[...], (((1,), (1,)), ((), ())),
        preferred_element_type=jnp.float32) + br_ref[...]


def _main_kernel(q_ref, sk_ref, v_ref, res_ref, o_ref):
    q = q_ref[...]
    sk = sk_ref[0]
    scores = jax.lax.dot_general(
        q, sk, (((1,), (1,)), ((), ())),
        preferred_element_type=jnp.float32)
    neg = jnp.float32(-jnp.inf)
    m = scores
    s_top = None
    denom = None
    for i in range(_TOP_K):
        cur = jnp.max(m, axis=1, keepdims=True)
        if i == 0:
            s_top = cur
            denom = jnp.ones_like(cur)
        else:
            denom = denom + jnp.exp(cur - s_top)
        m = jnp.where(m == cur, neg, m)
    ew16 = jnp.where(
        m == neg, jnp.exp(scores - s_top), 0.0).astype(jnp.bfloat16)
    out = jax.lax.dot_general(
        ew16, v_ref[0], (((1,), (0,)), ((), ())),
        preferred_element_type=jnp.float32)
    o_ref[0] = out / denom + res_ref[...]


def kernel(input, binding_keys, binding_values, Wq, bq, Wr, br):
    prefix = input.shape[:-1]
    d_in = input.shape[-1]
    n_slots, n_keys, k_dim = binding_keys.shape
    v_dim = binding_values.shape[-1]
    x = input.reshape(-1, d_in)
    bs = x.shape[0]
    values16 = binding_values.astype(jnp.bfloat16)

    sk = pl.pallas_call(
        _norm_kernel,
        grid=(n_slots,),
        in_specs=[pl.BlockSpec((1, n_keys, k_dim), lambda s: (s, 0, 0))],
        out_specs=pl.BlockSpec((1, n_keys, k_dim), lambda s: (s, 0, 0)),
        out_shape=jax.ShapeDtypeStruct((n_slots, n_keys, k_dim), jnp.bfloat16),
    )(binding_keys)

    tq = _row_tile(bs, 512)
    q16, res = pl.pallas_call(
        _qres_kernel,
        grid=(bs // tq,),
        in_specs=[
            pl.BlockSpec((tq, d_in), lambda r: (r, 0)),
            pl.BlockSpec((k_dim, d_in), lambda r: (0, 0)),
            pl.BlockSpec((1, k_dim), lambda r: (0, 0)),
            pl.BlockSpec((v_dim, k_dim), lambda r: (0, 0)),
            pl.BlockSpec((1, v_dim), lambda r: (0, 0)),
        ],
        out_specs=[
            pl.BlockSpec((tq, k_dim), lambda r: (r, 0)),
            pl.BlockSpec((tq, v_dim), lambda r: (r, 0)),
        ],
        out_shape=[
            jax.ShapeDtypeStruct((bs, k_dim), jnp.bfloat16),
            jax.ShapeDtypeStruct((bs, v_dim), jnp.float32),
        ],
    )(x, Wq, bq.reshape(1, -1), Wr, br.reshape(1, -1))

    tr = _row_tile(bs, 512)
    out = pl.pallas_call(
        _main_kernel,
        grid=(n_slots, bs // tr),
        in_specs=[
            pl.BlockSpec((tr, k_dim), lambda s, r: (r, 0)),
            pl.BlockSpec((1, n_keys, k_dim), lambda s, r: (s, 0, 0)),
            pl.BlockSpec((1, n_keys, v_dim), lambda s, r: (s, 0, 0)),
            pl.BlockSpec((tr, v_dim), lambda s, r: (r, 0)),
        ],
        out_specs=pl.BlockSpec((1, tr, v_dim), lambda s, r: (s, r, 0)),
        out_shape=jax.ShapeDtypeStruct((n_slots, bs, v_dim), jnp.float32),
    )(q16, sk, values16, res)

    return jnp.transpose(out, (1, 0, 2)).reshape(prefix + (n_slots, v_dim))
